# Initial kernel scaffold; baseline (speedup 1.0000x reference)
import jax
import jax.numpy as jnp
from jax.experimental import pallas as pl


def kernel(x):
    raise NotImplementedError("write your pallas kernel here")


# speedup vs baseline: 1.1491x; 1.1491x over previous
import jax
import jax.numpy as jnp
from jax import lax
from jax.experimental import pallas as pl
from jax.experimental.pallas import tpu as pltpu
from jax.experimental.pallas import tpu_sc as plsc

R = 128
N = 32768
L = 16
NC = 2
NS = 16
NW = NC * NS
ROWS_PER_W = R // NW
NV = N // L
CHUNK = N // L
RADIX = 256
HWORDS = RADIX * L


def _sort_body(x_hbm, out_hbm, keys, pa, pb, hist):
    cid = lax.axis_index("c")
    sid = lax.axis_index("s")
    wid = sid * NC + cid

    lane = lax.iota(jnp.int32, L)
    lane_base = lane * CHUNK
    ones = jnp.ones((L,), jnp.int32)
    zeros = jnp.zeros((L,), jnp.int32)

    for k in range(ROWS_PER_W):
        row = wid * ROWS_PER_W + k
        pltpu.sync_copy(x_hbm.at[row], keys)

        def _xform(i, carry):
            kv = keys[pl.ds(i * L, L)]
            m = jnp.where(kv < 0, jnp.int32(0), jnp.int32(0x7FFFFFFF))
            keys[pl.ds(i * L, L)] = kv ^ m
            return carry
        lax.fori_loop(0, NV, _xform, 0)

        for p in range(4):
            shift = 8 * p
            src = (None, pa, pb, pa)[p]
            dst = (pa, pb, pa, pb)[p]

            def _zero(j, carry):
                hist[pl.ds(j * L, L)] = zeros
                return carry
            lax.fori_loop(0, RADIX, _zero, 0)

            def _hist(i, carry):
                idx = lane_base + i
                pv = idx if src is None else plsc.load_gather(src, [idx])
                kv = plsc.load_gather(keys, [pv])
                digit = jnp.right_shift(kv, jnp.int32(shift)) & 255
                haddr = digit * L + lane
                plsc.addupdate_scatter(hist, [haddr], ones)
                return carry
            lax.fori_loop(0, NV, _hist, 0)

            def _scan(j, running):
                h = hist[pl.ds(j * L, L)]
                inc = plsc.cumsum(h)
                hist[pl.ds(j * L, L)] = inc - h + running
                return running + jnp.sum(h)
            lax.fori_loop(0, RADIX, _scan, jnp.int32(0))

            def _scat(i, carry):
                idx = lane_base + i
                pv = idx if src is None else plsc.load_gather(src, [idx])
                kv = plsc.load_gather(keys, [pv])
                digit = jnp.right_shift(kv, jnp.int32(shift)) & 255
                haddr = digit * L + lane
                old = plsc.load_gather(hist, [haddr])
                plsc.store_scatter(hist, [haddr], old + ones)
                plsc.store_scatter(dst, [old], pv)
                return carry
            lax.fori_loop(0, NV, _scat, 0)

        pltpu.sync_copy(pb, out_hbm.at[row])


def _argsort_desc(x_i32):
    mesh = plsc.VectorSubcoreMesh(
        core_axis_name="c", subcore_axis_name="s",
        num_cores=NC, num_subcores=NS)
    f = pl.kernel(
        _sort_body,
        out_type=jax.ShapeDtypeStruct((R, N), jnp.int32),
        mesh=mesh,
        compiler_params=pltpu.CompilerParams(needs_layout_passes=False),
        scratch_types=[
            pltpu.VMEM((N,), jnp.int32),
            pltpu.VMEM((N,), jnp.int32),
            pltpu.VMEM((N,), jnp.int32),
            pltpu.VMEM((HWORDS,), jnp.int32),
        ],
    )
    return f(x_i32)


def kernel(x):
    x_i32 = lax.bitcast_convert_type(x, jnp.int32)
    return _argsort_desc(x_i32).astype(jnp.int64)


# speedup vs baseline: 1.1705x; 1.0186x over previous
import jax
import jax.numpy as jnp
from jax import lax
from jax.experimental import pallas as pl
from jax.experimental.pallas import tpu as pltpu
from jax.experimental.pallas import tpu_sc as plsc

R = 128
N = 32768
L = 16
NC = 2
NS = 16
NW = NC * NS
ROWS_PER_W = R // NW
CHUNK = N // L
NB = 4
SUB = CHUNK // NB
RADIX = 256
HWORDS = RADIX * L


def _sort_body(x_hbm, out_hbm, keys, pa, pb, h0, h1, h2, h3):
    hb = (h0, h1, h2, h3)
    cid = lax.axis_index("c")
    sid = lax.axis_index("s")
    wid = sid * NC + cid

    lane = lax.iota(jnp.int32, L)
    ones = jnp.ones((L,), jnp.int32)
    zeros = jnp.zeros((L,), jnp.int32)
    bank_base = [lane * CHUNK + jnp.int32(b * SUB) for b in range(NB)]

    def digit_of(kv, shift):
        d = jnp.right_shift(kv, jnp.int32(shift)) & 255 if shift else kv & 255
        return d * L + lane

    for k in range(ROWS_PER_W):
        row = wid * ROWS_PER_W + k
        pltpu.sync_copy(x_hbm.at[row], keys)

        def _zero0(j, carry):
            for b in range(NB):
                hb[b][pl.ds(j * L, L)] = zeros
            return carry
        lax.fori_loop(0, RADIX, _zero0, 0)

        def _hist0(i, ivec):
            for b in range(NB):
                idx = bank_base[b] + ivec
                kv = plsc.load_gather(keys, [idx])
                kv = kv ^ jnp.where(kv < 0, jnp.int32(0), jnp.int32(0x7FFFFFFF))
                plsc.store_scatter(keys, [idx], kv)
                plsc.addupdate_scatter(hb[b], [digit_of(kv, 0)], ones)
            return ivec + 1
        lax.fori_loop(0, SUB, _hist0, zeros)

        for p in range(4):
            shift = 8 * p
            src = (None, pa, pb, pa)[p]
            dst = (pa, pb, pa, pb)[p]

            def _scan(j, running):
                hv = [hb[b][pl.ds(j * L, L)] for b in range(NB)]
                tot = hv[0]
                for b in range(1, NB):
                    tot = tot + hv[b]
                base = plsc.cumsum(tot) - tot + running
                acc = base
                for b in range(NB):
                    hb[b][pl.ds(j * L, L)] = acc
                    acc = acc + hv[b]
                return running + jnp.sum(tot)
            lax.fori_loop(0, RADIX, _scan, jnp.int32(0))

            if p < 3:
                nshift = shift + 8

                def _scat(i, ivec):
                    for b in range(NB):
                        idx = bank_base[b] + ivec
                        pv = idx if src is None else plsc.load_gather(src, [idx])
                        kv = plsc.load_gather(keys, [pv])
                        ha = digit_of(kv, shift)
                        old = plsc.load_gather(hb[b], [ha])
                        plsc.store_scatter(hb[b], [ha], old + ones)
                        plsc.store_scatter(dst, [old], pv)
                    return ivec + 1
                lax.fori_loop(0, SUB, _scat, zeros)

                def _zero(j, carry):
                    for b in range(NB):
                        hb[b][pl.ds(j * L, L)] = zeros
                    return carry
                lax.fori_loop(0, RADIX, _zero, 0)

                def _hist(i, ivec):
                    for b in range(NB):
                        idx = bank_base[b] + ivec
                        pv = plsc.load_gather(dst, [idx])
                        kv = plsc.load_gather(keys, [pv])
                        plsc.addupdate_scatter(hb[b], [digit_of(kv, nshift)], ones)
                    return ivec + 1
                lax.fori_loop(0, SUB, _hist, zeros)
            else:
                def _scat3(i, ivec):
                    for b in range(NB):
                        idx = bank_base[b] + ivec
                        pv = plsc.load_gather(src, [idx])
                        kv = plsc.load_gather(keys, [pv])
                        ha = digit_of(kv, shift)
                        old = plsc.load_gather(hb[b], [ha])
                        plsc.store_scatter(hb[b], [ha], old + ones)
                        plsc.store_scatter(dst, [old], pv)
                    return ivec + 1
                lax.fori_loop(0, SUB, _scat3, zeros)

        pltpu.sync_copy(pb, out_hbm.at[row])


def _argsort_desc(x_i32):
    mesh = plsc.VectorSubcoreMesh(
        core_axis_name="c", subcore_axis_name="s",
        num_cores=NC, num_subcores=NS)
    f = pl.kernel(
        _sort_body,
        out_type=jax.ShapeDtypeStruct((R, N), jnp.int32),
        mesh=mesh,
        compiler_params=pltpu.CompilerParams(needs_layout_passes=False),
        scratch_types=[
            pltpu.VMEM((N,), jnp.int32),
            pltpu.VMEM((N,), jnp.int32),
            pltpu.VMEM((N,), jnp.int32),
        ] + [pltpu.VMEM((HWORDS,), jnp.int32) for _ in range(NB)],
    )
    return f(x_i32)


def kernel(x):
    x_i32 = lax.bitcast_convert_type(x, jnp.int32)
    return _argsort_desc(x_i32).astype(jnp.int64)


# speedup vs baseline: 4.1791x; 3.5705x over previous
import jax
import jax.numpy as jnp
from jax import lax
from jax.experimental import pallas as pl
from jax.experimental.pallas import tpu as pltpu
from jax.experimental.pallas import tpu_sc as plsc

R = 128
N = 32768
L = 16
NC = 2
NS = 16
NW = NC * NS
ROWS_PER_W = R // NW
CHUNK = N // L
NB = 4
SUB = CHUNK // NB
RADIX = 256
HWORDS = RADIX * L
NPAD = N + L
B_ = range(NB)


def _sort_body(x_hbm, out_hbm, keys, pa, pb, h0, h1, h2, h3):
    hb = (h0, h1, h2, h3)
    cid = lax.axis_index("c")
    sid = lax.axis_index("s")
    wid = sid * NC + cid

    lane = lax.iota(jnp.int32, L)
    ones = jnp.ones((L,), jnp.int32)
    zeros = jnp.zeros((L,), jnp.int32)
    sbase = [lane * (CHUNK + 1) + jnp.int32(b * SUB) for b in B_]

    def phi(p):
        return p + jnp.right_shift(p, jnp.int32(11))

    def digit_of(kv, shift):
        d = jnp.right_shift(kv, jnp.int32(shift)) & 255 if shift else kv & 255
        return d * L + lane

    for k in range(ROWS_PER_W):
        row = wid * ROWS_PER_W + k
        pltpu.sync_copy(x_hbm.at[row], keys.at[pl.ds(0, N)])

        def _zero0(j, carry):
            for b in B_:
                hb[b][pl.ds(j * L, L)] = zeros
            return carry
        lax.fori_loop(0, RADIX, _zero0, 0)

        def _reloc(j, ivec):
            kv = keys[pl.ds((N // L - 1 - j) * L, L)]
            kv = kv ^ jnp.where(kv < 0, jnp.int32(0), jnp.int32(0x7FFFFFFF))
            plsc.store_scatter(keys, [phi(ivec)], kv)
            return ivec - L
        lax.fori_loop(0, N // L, _reloc, jnp.int32(N - L) + lane)

        def _hist0(i, ivec):
            idxs = [sbase[b] + ivec for b in B_]
            kvs = [plsc.load_gather(keys, [idxs[b]]) for b in B_]
            for b in B_:
                plsc.addupdate_scatter(hb[b], [digit_of(kvs[b], 0)], ones)
            return ivec + 1
        lax.fori_loop(0, SUB, _hist0, zeros)

        for p in range(4):
            shift = 8 * p
            src = (None, pa, pb, pa)[p]
            dst = (pa, pb, pa, pb)[p]
            last = p == 3

            def _scan(j, running):
                hv = [hb[b][pl.ds(j * L, L)] for b in B_]
                tot = hv[0]
                for b in range(1, NB):
                    tot = tot + hv[b]
                inc = plsc.cumsum(tot)
                acc = inc - tot + running
                for b in B_:
                    hb[b][pl.ds(j * L, L)] = acc
                    acc = acc + hv[b]
                return running + jnp.broadcast_to(inc[15], (L,))
            lax.fori_loop(0, RADIX, _scan, zeros)

            def _scat(i, ivec):
                idxs = [sbase[b] + ivec for b in B_]
                if src is None:
                    pvs = [(bb + ivec) for bb in
                           [lane * CHUNK + jnp.int32(b * SUB) for b in B_]]
                else:
                    pvs = [plsc.load_gather(src, [idxs[b]]) for b in B_]
                kvs = [plsc.load_gather(keys, [phi(pvs[b])]) for b in B_]
                has = [digit_of(kvs[b], shift) for b in B_]
                olds = [plsc.load_gather(hb[b], [has[b]]) for b in B_]
                for b in B_:
                    plsc.store_scatter(hb[b], [has[b]], olds[b] + ones)
                for b in B_:
                    plsc.store_scatter(
                        dst, [olds[b] if last else phi(olds[b])], pvs[b])
                return ivec + 1
            lax.fori_loop(0, SUB, _scat, zeros)

            if not last:
                nshift = shift + 8

                def _zero(j, carry):
                    for b in B_:
                        hb[b][pl.ds(j * L, L)] = zeros
                    return carry
                lax.fori_loop(0, RADIX, _zero, 0)

                def _hist(i, ivec):
                    idxs = [sbase[b] + ivec for b in B_]
                    pvs = [plsc.load_gather(dst, [idxs[b]]) for b in B_]
                    kvs = [plsc.load_gather(keys, [phi(pvs[b])]) for b in B_]
                    for b in B_:
                        plsc.addupdate_scatter(
                            hb[b], [digit_of(kvs[b], nshift)], ones)
                    return ivec + 1
                lax.fori_loop(0, SUB, _hist, zeros)

        pltpu.sync_copy(pb.at[pl.ds(0, N)], out_hbm.at[row])


def _argsort_desc(x_i32):
    mesh = plsc.VectorSubcoreMesh(
        core_axis_name="c", subcore_axis_name="s",
        num_cores=NC, num_subcores=NS)
    f = pl.kernel(
        _sort_body,
        out_type=jax.ShapeDtypeStruct((R, N), jnp.int32),
        mesh=mesh,
        compiler_params=pltpu.CompilerParams(needs_layout_passes=False),
        scratch_types=[
            pltpu.VMEM((NPAD,), jnp.int32),
            pltpu.VMEM((NPAD,), jnp.int32),
            pltpu.VMEM((NPAD,), jnp.int32),
        ] + [pltpu.VMEM((HWORDS,), jnp.int32) for _ in B_],
    )
    return f(x_i32)


def kernel(x):
    x_i32 = lax.bitcast_convert_type(x, jnp.int32)
    return _argsort_desc(x_i32).astype(jnp.int64)


# speedup vs baseline: 5.2929x; 1.2665x over previous
import jax
import jax.numpy as jnp
from jax import lax
from jax.experimental import pallas as pl
from jax.experimental.pallas import tpu as pltpu
from jax.experimental.pallas import tpu_sc as plsc

R = 128
N = 32768
L = 16
NC = 2
NS = 16
NW = NC * NS
ROWS_PER_W = R // NW
CHUNK = N // L
NB = 4
SUB = CHUNK // NB
RADIX = 256
HWORDS = RADIX * L
NPAD = N + L
B_ = range(NB)


def _sort_body(x_hbm, out_hbm, keys, pa, pb, h0, h1, h2, h3, sem):
    hb = (h0, h1, h2, h3)
    cid = lax.axis_index("c")
    sid = lax.axis_index("s")
    wid = sid * NC + cid

    lane = lax.iota(jnp.int32, L)
    ones = jnp.ones((L,), jnp.int32)
    zeros = jnp.zeros((L,), jnp.int32)
    sbase = [lane * (CHUNK + 1) + jnp.int32(b * SUB) for b in B_]
    pbase = [lane * CHUNK + jnp.int32(b * SUB) for b in B_]

    def phi(p):
        return p + jnp.right_shift(p, jnp.int32(11))

    def haddr_of(w):
        return (jnp.right_shift(w, jnp.int32(11)) & jnp.int32(0xFF0)) | lane

    out_dma = None
    for k in range(ROWS_PER_W):
        row = wid * ROWS_PER_W + k
        pltpu.sync_copy(x_hbm.at[row], keys.at[pl.ds(0, N)])

        def _zero0(j, carry):
            for b in B_:
                hb[b][pl.ds(j * L, L)] = zeros
            return carry
        lax.fori_loop(0, RADIX, _zero0, 0)

        def _reloc(j, ivec):
            kv = keys[pl.ds((N // L - 1 - j) * L, L)]
            kv = kv ^ jnp.where(kv < 0, jnp.int32(0), jnp.int32(0x7FFFFFFF))
            plsc.store_scatter(keys, [phi(ivec)], kv)
            return ivec - L
        lax.fori_loop(0, N // L, _reloc, jnp.int32(N - L) + lane)

        def _hist0(i, ivec):
            idxs = [sbase[b] + ivec for b in B_]
            kvs = [plsc.load_gather(keys, [idxs[b]]) for b in B_]
            for b in B_:
                plsc.addupdate_scatter(
                    hb[b], [((kvs[b] & 255) * L) | lane], ones)
            return ivec + 1
        lax.fori_loop(0, SUB, _hist0, zeros)

        for p in range(4):
            src = (None, pa, pb, pa)[p]
            dst = (pa, pb, pa, pb)[p]
            last = p == 3

            def _scan(j, running):
                hv = [hb[b][pl.ds(j * L, L)] for b in B_]
                tot = hv[0]
                for b in range(1, NB):
                    tot = tot + hv[b]
                inc = plsc.cumsum(tot)
                acc = inc - tot + running
                for b in B_:
                    hb[b][pl.ds(j * L, L)] = acc
                    acc = acc + hv[b]
                return running + jnp.broadcast_to(inc[15], (L,))
            lax.fori_loop(0, RADIX, _scan, zeros)

            if p == 1 and out_dma is not None:
                out_dma.wait()
                out_dma = None

            def _scat(i, ivec):
                if p == 0:
                    idxs = [sbase[b] + ivec for b in B_]
                    kvs = [plsc.load_gather(keys, [idxs[b]]) for b in B_]
                    pvs = [pbase[b] + ivec for b in B_]
                    has = [((kvs[b] & 255) * L) | lane for b in B_]
                    words = [pvs[b] | ((kvs[b] & jnp.int32(0xFF00)) * 128)
                             for b in B_]
                else:
                    ws = [plsc.load_gather(src, [sbase[b] + ivec]) for b in B_]
                    pvs = [w & jnp.int32(0x7FFF) for w in ws]
                    has = [haddr_of(w) for w in ws]
                    if last:
                        words = pvs
                    else:
                        kvs = [plsc.load_gather(keys, [phi(pvs[b])])
                               for b in B_]
                        nd = [jnp.right_shift(kvs[b], jnp.int32(8 * p + 8))
                              & 255 for b in B_]
                        words = [pvs[b] | (nd[b] * 32768) for b in B_]
                olds = [plsc.load_gather(hb[b], [has[b]]) for b in B_]
                for b in B_:
                    plsc.store_scatter(hb[b], [has[b]], olds[b] + ones)
                for b in B_:
                    plsc.store_scatter(
                        dst, [olds[b] if last else phi(olds[b])], words[b])
                return ivec + 1
            lax.fori_loop(0, SUB, _scat, zeros)

            if not last:
                def _zero(j, carry):
                    for b in B_:
                        hb[b][pl.ds(j * L, L)] = zeros
                    return carry
                lax.fori_loop(0, RADIX, _zero, 0)

                def _hist(i, ivec):
                    ws = [plsc.load_gather(dst, [sbase[b] + ivec]) for b in B_]
                    for b in B_:
                        plsc.addupdate_scatter(hb[b], [haddr_of(ws[b])], ones)
                    return ivec + 1
                lax.fori_loop(0, SUB, _hist, zeros)

        out_dma = pltpu.make_async_copy(
            pb.at[pl.ds(0, N)], out_hbm.at[row], sem)
        out_dma.start()

    out_dma.wait()


def _argsort_desc(x_i32):
    mesh = plsc.VectorSubcoreMesh(
        core_axis_name="c", subcore_axis_name="s",
        num_cores=NC, num_subcores=NS)
    f = pl.kernel(
        _sort_body,
        out_type=jax.ShapeDtypeStruct((R, N), jnp.int32),
        mesh=mesh,
        compiler_params=pltpu.CompilerParams(needs_layout_passes=False),
        scratch_types=[
            pltpu.VMEM((NPAD,), jnp.int32),
            pltpu.VMEM((NPAD,), jnp.int32),
            pltpu.VMEM((NPAD,), jnp.int32),
            pltpu.VMEM((HWORDS,), jnp.int32),
            pltpu.VMEM((HWORDS,), jnp.int32),
            pltpu.VMEM((HWORDS,), jnp.int32),
            pltpu.VMEM((HWORDS,), jnp.int32),
            pltpu.SemaphoreType.DMA,
        ],
    )
    return f(x_i32)


def kernel(x):
    x_i32 = lax.bitcast_convert_type(x, jnp.int32)
    return _argsort_desc(x_i32).astype(jnp.int64)


# speedup vs baseline: 7.0823x; 1.3381x over previous
import jax
import jax.numpy as jnp
from jax import lax
from jax.experimental import pallas as pl
from jax.experimental.pallas import tpu as pltpu
from jax.experimental.pallas import tpu_sc as plsc

R = 128
N = 32768
L = 16
NC = 2
NS = 16
NW = NC * NS
ROWS_PER_W = R // NW
CHUNK = N // L
NB = 4
SUB = CHUNK // NB
RADIX = 256
HWORDS = RADIX * L
NPAD = N + L
B_ = range(NB)


def _sort_body(x_hbm, out_hbm, keys, pa, pb, h0, h1, h2, h3, sem_in, sem_out):
    hb = (h0, h1, h2, h3)
    cid = lax.axis_index("c")
    sid = lax.axis_index("s")
    wid = sid * NC + cid

    lane = lax.iota(jnp.int32, L)
    ones = jnp.ones((L,), jnp.int32)
    zeros = jnp.zeros((L,), jnp.int32)
    sbase = [lane * (CHUNK + 1) + jnp.int32(b * SUB) for b in B_]
    pbase = [lane * CHUNK + jnp.int32(b * SUB) for b in B_]

    def phi(p):
        return p + jnp.right_shift(p, jnp.int32(11))

    def haddr_of(w):
        return (jnp.right_shift(w, jnp.int32(11)) & jnp.int32(0xFF0)) | lane

    out_dma = None
    in_dma = None
    for k in range(ROWS_PER_W):
        row = wid * ROWS_PER_W + k
        if in_dma is None:
            pltpu.sync_copy(x_hbm.at[row], keys.at[pl.ds(0, N)])
        else:
            in_dma.wait()
            in_dma = None

        @plsc.parallel_loop(0, RADIX, unroll=4)
        def _zero0(j):
            for b in B_:
                hb[b][pl.ds(j * L, L)] = zeros

        def _reloc(j, ivec):
            kv = keys[pl.ds((N // L - 1 - j) * L, L)]
            kv = kv ^ jnp.where(kv < 0, jnp.int32(0), jnp.int32(0x7FFFFFFF))
            plsc.store_scatter(keys, [phi(ivec)], kv)
            return ivec - L
        lax.fori_loop(0, N // L, _reloc, jnp.int32(N - L) + lane)

        @plsc.parallel_loop(0, SUB, carry=zeros, unroll=2)
        def _hist0(i, ivec):
            idxs = [sbase[b] + ivec for b in B_]
            kvs = [plsc.load_gather(keys, [idxs[b]]) for b in B_]
            for b in B_:
                plsc.addupdate_scatter(
                    hb[b], [((kvs[b] & 255) * L) | lane], ones)
            return ivec + 1

        for p in range(4):
            src = (None, pa, pb, pa)[p]
            dst = (pa, pb, pa, pb)[p]
            last = p == 3

            @plsc.parallel_loop(0, RADIX, carry=zeros, unroll=2)
            def _scan(j, running):
                hv = [hb[b][pl.ds(j * L, L)] for b in B_]
                tot = hv[0]
                for b in range(1, NB):
                    tot = tot + hv[b]
                inc = plsc.cumsum(tot)
                acc = inc - tot + running
                for b in B_:
                    hb[b][pl.ds(j * L, L)] = acc
                    acc = acc + hv[b]
                return running + jnp.broadcast_to(inc[15], (L,))

            if p == 1 and out_dma is not None:
                out_dma.wait()
                out_dma = None

            def _scat_once(ivec):
                if p == 0:
                    idxs = [sbase[b] + ivec for b in B_]
                    kvs = [plsc.load_gather(keys, [idxs[b]]) for b in B_]
                    pvs = [pbase[b] + ivec for b in B_]
                    has = [((kvs[b] & 255) * L) | lane for b in B_]
                    words = [pvs[b] | ((kvs[b] & jnp.int32(0xFF00)) * 128)
                             for b in B_]
                else:
                    ws = [plsc.load_gather(src, [sbase[b] + ivec]) for b in B_]
                    pvs = [w & jnp.int32(0x7FFF) for w in ws]
                    has = [haddr_of(w) for w in ws]
                    if last:
                        words = pvs
                    else:
                        kvs = [plsc.load_gather(keys, [phi(pvs[b])])
                               for b in B_]
                        if p == 1:
                            nd = [jnp.right_shift(kv, jnp.int32(1))
                                  & jnp.int32(0x7F8000) for kv in kvs]
                        else:
                            nd = [jnp.right_shift(kv, jnp.int32(9))
                                  & jnp.int32(0x7F8000) for kv in kvs]
                        words = [pvs[b] | nd[b] for b in B_]
                olds = [plsc.load_gather(hb[b], [has[b]]) for b in B_]
                for b in B_:
                    plsc.store_scatter(hb[b], [has[b]], olds[b] + ones)
                for b in B_:
                    plsc.store_scatter(
                        dst, [olds[b] if last else phi(olds[b])], words[b])

            def _scat(i, ivec):
                _scat_once(ivec)
                _scat_once(ivec + 1)
                return ivec + 2
            lax.fori_loop(0, SUB // 2, _scat, zeros)

            if p == 2 and k + 1 < ROWS_PER_W:
                in_dma = pltpu.make_async_copy(
                    x_hbm.at[row + 1], keys.at[pl.ds(0, N)], sem_in)
                in_dma.start()

            if not last:
                @plsc.parallel_loop(0, RADIX, unroll=4)
                def _zero(j):
                    for b in B_:
                        hb[b][pl.ds(j * L, L)] = zeros

                @plsc.parallel_loop(0, SUB, carry=zeros, unroll=2)
                def _hist(i, ivec):
                    ws = [plsc.load_gather(dst, [sbase[b] + ivec]) for b in B_]
                    for b in B_:
                        plsc.addupdate_scatter(hb[b], [haddr_of(ws[b])], ones)
                    return ivec + 1

        out_dma = pltpu.make_async_copy(
            pb.at[pl.ds(0, N)], out_hbm.at[row], sem_out)
        out_dma.start()

    out_dma.wait()


def _argsort_desc(x_i32):
    mesh = plsc.VectorSubcoreMesh(
        core_axis_name="c", subcore_axis_name="s",
        num_cores=NC, num_subcores=NS)
    f = pl.kernel(
        _sort_body,
        out_type=jax.ShapeDtypeStruct((R, N), jnp.int32),
        mesh=mesh,
        compiler_params=pltpu.CompilerParams(needs_layout_passes=False),
        scratch_types=[
            pltpu.VMEM((NPAD,), jnp.int32),
            pltpu.VMEM((NPAD,), jnp.int32),
            pltpu.VMEM((NPAD,), jnp.int32),
            pltpu.VMEM((HWORDS,), jnp.int32),
            pltpu.VMEM((HWORDS,), jnp.int32),
            pltpu.VMEM((HWORDS,), jnp.int32),
            pltpu.VMEM((HWORDS,), jnp.int32),
            pltpu.SemaphoreType.DMA,
            pltpu.SemaphoreType.DMA,
        ],
    )
    return f(x_i32)


def kernel(x):
    x_i32 = lax.bitcast_convert_type(x, jnp.int32)
    return _argsort_desc(x_i32).astype(jnp.int64)


# speedup vs baseline: 7.1192x; 1.0052x over previous
import jax
import jax.numpy as jnp
from jax import lax
from jax.experimental import pallas as pl
from jax.experimental.pallas import tpu as pltpu
from jax.experimental.pallas import tpu_sc as plsc

R = 128
N = 32768
L = 16
NC = 2
NS = 16
NW = NC * NS
ROWS_PER_W = R // NW
CHUNK = N // L
NB = 4
SUB = CHUNK // NB
RADIX = 256
HWORDS = RADIX * L
NPAD = N + L
B_ = range(NB)


def _sort_body(x_hbm, out_hbm, keys, pa, pb, h0, h1, h2, h3, sem_in, sem_out):
    hb = (h0, h1, h2, h3)
    cid = lax.axis_index("c")
    sid = lax.axis_index("s")
    wid = sid * NC + cid

    lane = lax.iota(jnp.int32, L)
    ones = jnp.ones((L,), jnp.int32)
    zeros = jnp.zeros((L,), jnp.int32)
    sbase = [lane * (CHUNK + 1) + jnp.int32(b * SUB) for b in B_]
    pbase = [lane * CHUNK + jnp.int32(b * SUB) for b in B_]

    def phi(p):
        return p + jnp.right_shift(p, jnp.int32(11))

    def haddr_of(w):
        return (jnp.right_shift(w, jnp.int32(11)) & jnp.int32(0xFF0)) | lane

    out_dma = None
    in_dma = None
    for k in range(ROWS_PER_W):
        row = wid * ROWS_PER_W + k
        if in_dma is None:
            pltpu.sync_copy(x_hbm.at[row], keys.at[pl.ds(0, N)])
        else:
            in_dma.wait()
            in_dma = None

        @plsc.parallel_loop(0, RADIX, unroll=4)
        def _zero0(j):
            for b in B_:
                hb[b][pl.ds(j * L, L)] = zeros

        def _reloc(j, ivec):
            for u in range(4):
                iv = ivec - u * L
                kv = keys[pl.ds((N // L - 1 - (j * 4 + u)) * L, L)]
                kv = kv ^ jnp.where(kv < 0, jnp.int32(0), jnp.int32(0x7FFFFFFF))
                plsc.store_scatter(keys, [phi(iv)], kv)
            return ivec - 4 * L
        lax.fori_loop(0, N // L // 4, _reloc, jnp.int32(N - L) + lane)

        @plsc.parallel_loop(0, SUB, carry=zeros, unroll=2)
        def _hist0(i, ivec):
            idxs = [sbase[b] + ivec for b in B_]
            kvs = [plsc.load_gather(keys, [idxs[b]]) for b in B_]
            for b in B_:
                plsc.addupdate_scatter(
                    hb[b], [((kvs[b] & 255) * L) | lane], ones)
            return ivec + 1

        for p in range(4):
            src = (None, pa, pb, pa)[p]
            dst = (pa, pb, pa, pb)[p]
            last = p == 3

            @plsc.parallel_loop(0, RADIX, carry=zeros, unroll=2)
            def _scan(j, running):
                hv = [hb[b][pl.ds(j * L, L)] for b in B_]
                tot = hv[0]
                for b in range(1, NB):
                    tot = tot + hv[b]
                inc = plsc.cumsum(tot)
                acc = inc - tot + running
                for b in B_:
                    hb[b][pl.ds(j * L, L)] = acc
                    acc = acc + hv[b]
                return running + jnp.broadcast_to(inc[15], (L,))

            if p == 1 and out_dma is not None:
                out_dma.wait()
                out_dma = None

            def _scat_once(ivec):
                if p == 0:
                    idxs = [sbase[b] + ivec for b in B_]
                    kvs = [plsc.load_gather(keys, [idxs[b]]) for b in B_]
                    pvs = [pbase[b] + ivec for b in B_]
                    has = [((kvs[b] & 255) * L) | lane for b in B_]
                    words = [pvs[b] | ((kvs[b] & jnp.int32(0xFF00)) * 128)
                             for b in B_]
                else:
                    ws = [plsc.load_gather(src, [sbase[b] + ivec]) for b in B_]
                    pvs = [w & jnp.int32(0x7FFF) for w in ws]
                    has = [haddr_of(w) for w in ws]
                    if last:
                        words = pvs
                    else:
                        kvs = [plsc.load_gather(keys, [phi(pvs[b])])
                               for b in B_]
                        if p == 1:
                            nd = [jnp.right_shift(kv, jnp.int32(1))
                                  & jnp.int32(0x7F8000) for kv in kvs]
                        else:
                            nd = [jnp.right_shift(kv, jnp.int32(9))
                                  & jnp.int32(0x7F8000) for kv in kvs]
                        words = [pvs[b] | nd[b] for b in B_]
                olds = [plsc.load_gather(hb[b], [has[b]]) for b in B_]
                for b in B_:
                    plsc.store_scatter(hb[b], [has[b]], olds[b] + ones)
                for b in B_:
                    plsc.store_scatter(
                        dst, [olds[b] if last else phi(olds[b])], words[b])

            def _scat(i, ivec):
                _scat_once(ivec)
                _scat_once(ivec + 1)
                return ivec + 2
            lax.fori_loop(0, SUB // 2, _scat, zeros)

            if p == 2 and k + 1 < ROWS_PER_W:
                in_dma = pltpu.make_async_copy(
                    x_hbm.at[row + 1], keys.at[pl.ds(0, N)], sem_in)
                in_dma.start()

            if not last:
                @plsc.parallel_loop(0, RADIX, unroll=4)
                def _zero(j):
                    for b in B_:
                        hb[b][pl.ds(j * L, L)] = zeros

                @plsc.parallel_loop(0, SUB, carry=zeros, unroll=2)
                def _hist(i, ivec):
                    ws = [plsc.load_gather(dst, [sbase[b] + ivec]) for b in B_]
                    for b in B_:
                        plsc.addupdate_scatter(hb[b], [haddr_of(ws[b])], ones)
                    return ivec + 1

        out_dma = pltpu.make_async_copy(
            pb.at[pl.ds(0, N)], out_hbm.at[row], sem_out)
        out_dma.start()

    out_dma.wait()


def _argsort_desc(x_i32):
    mesh = plsc.VectorSubcoreMesh(
        core_axis_name="c", subcore_axis_name="s",
        num_cores=NC, num_subcores=NS)
    f = pl.kernel(
        _sort_body,
        out_type=jax.ShapeDtypeStruct((R, N), jnp.int32),
        mesh=mesh,
        compiler_params=pltpu.CompilerParams(needs_layout_passes=False),
        scratch_types=[
            pltpu.VMEM((NPAD,), jnp.int32),
            pltpu.VMEM((NPAD,), jnp.int32),
            pltpu.VMEM((NPAD,), jnp.int32),
            pltpu.VMEM((HWORDS,), jnp.int32),
            pltpu.VMEM((HWORDS,), jnp.int32),
            pltpu.VMEM((HWORDS,), jnp.int32),
            pltpu.VMEM((HWORDS,), jnp.int32),
            pltpu.SemaphoreType.DMA,
            pltpu.SemaphoreType.DMA,
        ],
    )
    return f(x_i32)


def kernel(x):
    x_i32 = lax.bitcast_convert_type(x, jnp.int32)
    return _argsort_desc(x_i32).astype(jnp.int64)


# speedup vs baseline: 7.3549x; 1.0331x over previous
import jax
import jax.numpy as jnp
from jax import lax
from jax.experimental import pallas as pl
from jax.experimental.pallas import tpu as pltpu
from jax.experimental.pallas import tpu_sc as plsc

R = 128
N = 32768
L = 16
NC = 2
NS = 16
NW = NC * NS
ROWS_PER_W = R // NW
CHUNK = N // L
NB = 4
SUB = CHUNK // NB
RADIX = 256
HWORDS = RADIX * L
NPAD = N + L
B_ = range(NB)


def _sort_body(x_hbm, out_hbm, keys, pa, pb, h0, h1, h2, h3, sem_in, sem_out):
    hb = (h0, h1, h2, h3)
    cid = lax.axis_index("c")
    sid = lax.axis_index("s")
    wid = sid * NC + cid

    lane = lax.iota(jnp.int32, L)
    ones = jnp.ones((L,), jnp.int32)
    zeros = jnp.zeros((L,), jnp.int32)
    sbase = [lane * (CHUNK + 1) + jnp.int32(b * SUB) for b in B_]
    pbase = [lane * CHUNK + jnp.int32(b * SUB) for b in B_]

    def phi(p):
        return p + jnp.right_shift(p, jnp.int32(11))

    def haddr_of(w):
        return (jnp.right_shift(w, jnp.int32(11)) & jnp.int32(0xFF0)) | lane

    out_dma = None
    in_dma = None
    for k in range(ROWS_PER_W):
        row = wid * ROWS_PER_W + k
        if in_dma is None:
            pltpu.sync_copy(x_hbm.at[row], keys.at[pl.ds(0, N)])
        else:
            in_dma.wait()
            in_dma = None

        @plsc.parallel_loop(0, RADIX, unroll=4)
        def _zero0(j):
            for b in B_:
                hb[b][pl.ds(j * L, L)] = zeros

        def _reloc(j, ivec):
            for u in range(4):
                iv = ivec - u * L
                kv = keys[pl.ds((N // L - 1 - (j * 4 + u)) * L, L)]
                kv = kv ^ jnp.where(kv < 0, jnp.int32(0), jnp.int32(0x7FFFFFFF))
                plsc.store_scatter(keys, [phi(iv)], kv)
            return ivec - 4 * L
        lax.fori_loop(0, N // L // 4, _reloc, jnp.int32(N - L) + lane)

        @plsc.parallel_loop(0, SUB, carry=zeros, unroll=2)
        def _hist0(i, ivec):
            kvs = [plsc.load_gather(keys, [sbase[b] + ivec]) for b in B_]
            for b in B_:
                plsc.addupdate_scatter(
                    hb[b], [((kvs[b] & 255) * L) | lane], ones)
            return ivec + 1

        for p in range(4):
            last = p == 3

            @plsc.parallel_loop(0, RADIX, carry=zeros, unroll=2)
            def _scan(j, running):
                hv = [hb[b][pl.ds(j * L, L)] for b in B_]
                tot = hv[0]
                for b in range(1, NB):
                    tot = tot + hv[b]
                inc = plsc.cumsum(tot)
                acc = inc - tot + running
                for b in B_:
                    hb[b][pl.ds(j * L, L)] = acc
                    acc = acc + hv[b]
                return running + jnp.broadcast_to(inc[15], (L,))

            if p == 0 and out_dma is not None:
                out_dma.wait()
                out_dma = None

            def _scat_once(ivec):
                if p == 0:
                    kvs = [plsc.load_gather(keys, [sbase[b] + ivec])
                           for b in B_]
                    has = [((kvs[b] & 255) * L) | lane for b in B_]
                    words = [(pbase[b] + ivec)
                             | ((kvs[b] & jnp.int32(0xFF00)) * 128)
                             for b in B_]
                    khis = [jnp.right_shift(kvs[b], jnp.int32(16))
                            & jnp.int32(0xFFFF) for b in B_]
                else:
                    if p == 1:
                        ws = [plsc.load_gather(pa, [sbase[b] + ivec])
                              for b in B_]
                        khs = [plsc.load_gather(pb, [sbase[b] + ivec])
                               for b in B_]
                        words = [(w & jnp.int32(0x7FFF)) | (kh * 32768)
                                 for w, kh in zip(ws, khs)]
                    elif p == 2:
                        ws = [plsc.load_gather(keys, [sbase[b] + ivec])
                              for b in B_]
                        words = [(w & jnp.int32(0x7FFF))
                                 | (jnp.right_shift(w, jnp.int32(8))
                                    & jnp.int32(0x7F8000)) for w in ws]
                    else:
                        ws = [plsc.load_gather(pa, [sbase[b] + ivec])
                              for b in B_]
                        words = [w & jnp.int32(0x7FFF) for w in ws]
                    has = [haddr_of(w) for w in ws]
                olds = [plsc.load_gather(hb[b], [has[b]]) for b in B_]
                for b in B_:
                    plsc.store_scatter(hb[b], [has[b]], olds[b] + ones)
                if p == 0:
                    pos = [phi(o) for o in olds]
                    for b in B_:
                        plsc.store_scatter(pa, [pos[b]], words[b])
                    for b in B_:
                        plsc.store_scatter(pb, [pos[b]], khis[b])
                elif p == 1:
                    for b in B_:
                        plsc.store_scatter(keys, [phi(olds[b])], words[b])
                elif p == 2:
                    for b in B_:
                        plsc.store_scatter(pa, [phi(olds[b])], words[b])
                else:
                    for b in B_:
                        plsc.store_scatter(pb, [olds[b]], words[b])

            def _scat(i, ivec):
                _scat_once(ivec)
                _scat_once(ivec + 1)
                _scat_once(ivec + 2)
                _scat_once(ivec + 3)
                return ivec + 4
            lax.fori_loop(0, SUB // 4, _scat, zeros)

            if p == 2 and k + 1 < ROWS_PER_W:
                in_dma = pltpu.make_async_copy(
                    x_hbm.at[row + 1], keys.at[pl.ds(0, N)], sem_in)
                in_dma.start()

            if not last:
                nxt = (pa, keys, pa)[p]

                @plsc.parallel_loop(0, RADIX, unroll=4)
                def _zero(j):
                    for b in B_:
                        hb[b][pl.ds(j * L, L)] = zeros

                @plsc.parallel_loop(0, SUB, carry=zeros, unroll=2)
                def _hist(i, ivec):
                    ws = [plsc.load_gather(nxt, [sbase[b] + ivec]) for b in B_]
                    for b in B_:
                        plsc.addupdate_scatter(hb[b], [haddr_of(ws[b])], ones)
                    return ivec + 1

        out_dma = pltpu.make_async_copy(
            pb.at[pl.ds(0, N)], out_hbm.at[row], sem_out)
        out_dma.start()

    out_dma.wait()


def _argsort_desc(x_i32):
    mesh = plsc.VectorSubcoreMesh(
        core_axis_name="c", subcore_axis_name="s",
        num_cores=NC, num_subcores=NS)
    f = pl.kernel(
        _sort_body,
        out_type=jax.ShapeDtypeStruct((R, N), jnp.int32),
        mesh=mesh,
        compiler_params=pltpu.CompilerParams(needs_layout_passes=False),
        scratch_types=[
            pltpu.VMEM((NPAD,), jnp.int32),
            pltpu.VMEM((NPAD,), jnp.int32),
            pltpu.VMEM((NPAD,), jnp.int32),
            pltpu.VMEM((HWORDS,), jnp.int32),
            pltpu.VMEM((HWORDS,), jnp.int32),
            pltpu.VMEM((HWORDS,), jnp.int32),
            pltpu.VMEM((HWORDS,), jnp.int32),
            pltpu.SemaphoreType.DMA,
            pltpu.SemaphoreType.DMA,
        ],
    )
    return f(x_i32)


def kernel(x):
    x_i32 = lax.bitcast_convert_type(x, jnp.int32)
    return _argsort_desc(x_i32).astype(jnp.int64)
